# baseline (speedup 1.0000x reference)
import functools

import jax
import jax.numpy as jnp
from jax import lax
from jax.experimental import pallas as pl
from jax.experimental.pallas import tpu as pltpu
from jax.experimental.pallas import tpu_sc as plsc

BATCH = 16384
EMBED_DIM = 16

NUM_CORES = 2
NUM_SUBCORES = 16
NUM_WORKERS = NUM_CORES * NUM_SUBCORES
B_PER_W = BATCH // NUM_WORKERS
CHUNK = 128
NCHUNK = B_PER_W // CHUNK

_mesh = plsc.VectorSubcoreMesh(core_axis_name="c", subcore_axis_name="s")


@functools.partial(
    pl.kernel,
    mesh=_mesh,
    out_type=jax.ShapeDtypeStruct((BATCH, EMBED_DIM), jnp.float32),
    scratch_types=[
        pltpu.VMEM((NCHUNK, CHUNK), jnp.int32),
        pltpu.VMEM((NCHUNK, CHUNK, EMBED_DIM), jnp.float32),
        pltpu.SemaphoreType.DMA,
    ],
    compiler_params=pltpu.CompilerParams(use_tc_tiling_on_sc=False),
)
def _embed_gather(idx_hbm, table_hbm, out_hbm, idx_v, rows_v, sem):
    wid = lax.axis_index("s") * NUM_CORES + lax.axis_index("c")
    base = wid * B_PER_W
    pltpu.sync_copy(idx_hbm.at[wid], idx_v)
    copies = [
        pltpu.async_copy(table_hbm.at[idx_v.at[j]], rows_v.at[j], sem)
        for j in range(NCHUNK)
    ]
    for c in copies:
        c.wait()
    for j in range(NCHUNK):
        pltpu.sync_copy(rows_v.at[j], out_hbm.at[pl.ds(base + j * CHUNK, CHUNK)])


def kernel(inputs, table):
    idx = inputs.astype(jnp.int32).reshape(NUM_WORKERS, NCHUNK, CHUNK)
    return _embed_gather(idx, table)


# speedup vs baseline: 4.9972x; 4.9972x over previous
import functools

import jax
import jax.numpy as jnp
from jax import lax
from jax.experimental import pallas as pl
from jax.experimental.pallas import tpu as pltpu
from jax.experimental.pallas import tpu_sc as plsc

BATCH = 16384
EMBED_DIM = 16
LANES = 16

NUM_CORES = 2
NUM_SUBCORES = 16
NUM_WORKERS = NUM_CORES * NUM_SUBCORES
B_PER_W = BATCH // NUM_WORKERS
GROUPS = B_PER_W // LANES

_mesh = plsc.VectorSubcoreMesh(core_axis_name="c", subcore_axis_name="s")


@functools.partial(
    pl.kernel,
    mesh=_mesh,
    out_type=jax.ShapeDtypeStruct((EMBED_DIM, BATCH), jnp.float32),
    scratch_types=[
        pltpu.VMEM((B_PER_W,), jnp.int32),
        pltpu.VMEM((LANES * EMBED_DIM, 128), jnp.float32),
        pltpu.VMEM((EMBED_DIM, B_PER_W), jnp.float32),
        pltpu.SemaphoreType.DMA,
        pltpu.SemaphoreType.DMA,
    ],
)
def _embed_gather_t(idx_hbm, tab_t_hbm, out_t_hbm, idx_v, stage_v, out_v,
                    sem_i, sem):
    wid = lax.axis_index("s") * NUM_CORES + lax.axis_index("c")
    base = wid * B_PER_W
    pltpu.async_copy(idx_hbm.at[pl.ds(base, B_PER_W)], idx_v, sem_i).wait()
    lane_ids = lax.iota(jnp.int32, LANES)

    def body(g, _):
        vec = idx_v[pl.ds(g * LANES, LANES)]
        cols = lax.bitwise_and(vec, 127)
        blocks = lax.shift_right_logical(vec, 7)
        copies = []
        for k in range(LANES):
            a = pl.multiple_of(blocks[k] * 128, 128)
            copies.append(
                pltpu.async_copy(
                    tab_t_hbm.at[:, pl.ds(a, 128)],
                    stage_v.at[pl.ds(k * EMBED_DIM, EMBED_DIM), :],
                    sem,
                )
            )
        for cp in copies:
            cp.wait()
        cq = lax.bitwise_and(cols, 0x70)
        cr = lax.bitwise_and(cols, 0xF)
        for j in range(EMBED_DIM):
            acc = jnp.zeros((LANES,), jnp.float32)
            for k in range(LANES):
                v16 = stage_v[k * EMBED_DIM + j, pl.ds(cq[k], LANES)]
                val = v16[jnp.broadcast_to(cr[k], (LANES,))]
                acc = jnp.where(lane_ids == k, val, acc)
            out_v[j, pl.ds(g * LANES, LANES)] = acc
        return 0

    lax.fori_loop(0, GROUPS, body, 0)
    pltpu.async_copy(out_v, out_t_hbm.at[:, pl.ds(base, B_PER_W)], sem).wait()


def kernel(inputs, table):
    idx = inputs.astype(jnp.int32)
    out_t = _embed_gather_t(idx, table.T)
    return out_t.T


# speedup vs baseline: 5.9331x; 1.1873x over previous
import functools

import jax
import jax.numpy as jnp
from jax import lax
from jax.experimental import pallas as pl
from jax.experimental.pallas import tpu as pltpu
from jax.experimental.pallas import tpu_sc as plsc

BATCH = 16384
EMBED_DIM = 16
LANES = 16

NUM_CORES = 2
NUM_SUBCORES = 16
NUM_WORKERS = NUM_CORES * NUM_SUBCORES
B_PER_W = BATCH // NUM_WORKERS
GROUPS = B_PER_W // LANES

_mesh = plsc.VectorSubcoreMesh(core_axis_name="c", subcore_axis_name="s")


@functools.partial(
    pl.kernel,
    mesh=_mesh,
    out_type=jax.ShapeDtypeStruct((EMBED_DIM, BATCH), jnp.float32),
    scratch_types=[
        pltpu.VMEM((B_PER_W,), jnp.int32),
        pltpu.VMEM((2, LANES * EMBED_DIM, 128), jnp.float32),
        pltpu.VMEM((EMBED_DIM, B_PER_W), jnp.float32),
        pltpu.SemaphoreType.DMA,
        pltpu.SemaphoreType.DMA,
        pltpu.SemaphoreType.DMA,
    ],
)
def _embed_gather_t(idx_hbm, tab_t_hbm, out_t_hbm, idx_v, stage_v, out_v,
                    sem_i, sem_a, sem_b):
    wid = lax.axis_index("s") * NUM_CORES + lax.axis_index("c")
    base = wid * B_PER_W
    pltpu.async_copy(idx_hbm.at[pl.ds(base, B_PER_W)], idx_v, sem_i).wait()
    lane_ids = lax.iota(jnp.int32, LANES)
    sems = (sem_a, sem_b)

    def fire(g, buf):
        vec = idx_v[pl.ds(g * LANES, LANES)]
        blocks = lax.shift_right_logical(vec, 7)
        for k in range(LANES):
            a = pl.multiple_of(blocks[k] * 128, 128)
            pltpu.async_copy(
                tab_t_hbm.at[:, pl.ds(a, 128)],
                stage_v.at[buf, pl.ds(k * EMBED_DIM, EMBED_DIM), :],
                sems[buf],
            )

    def drain_extract(g, buf):
        for k in range(LANES):
            pltpu.make_async_copy(
                tab_t_hbm.at[:, pl.ds(0, 128)],
                stage_v.at[buf, pl.ds(k * EMBED_DIM, EMBED_DIM), :],
                sems[buf],
            ).wait()
        vec = idx_v[pl.ds(g * LANES, LANES)]
        cols = lax.bitwise_and(vec, 127)
        cq = lax.bitwise_and(cols, 0x70)
        cr = lax.bitwise_and(cols, 0xF)
        for j in range(EMBED_DIM):
            acc = jnp.zeros((LANES,), jnp.float32)
            for k in range(LANES):
                v16 = stage_v[buf, k * EMBED_DIM + j, pl.ds(cq[k], LANES)]
                val = v16[jnp.broadcast_to(cr[k], (LANES,))]
                acc = jnp.where(lane_ids == k, val, acc)
            out_v[j, pl.ds(g * LANES, LANES)] = acc

    fire(0, 0)

    def body(gg, _):
        g = 2 * gg
        fire(g + 1, 1)
        drain_extract(g, 0)

        @pl.when(g + 2 < GROUPS)
        def _():
            fire(g + 2, 0)

        drain_extract(g + 1, 1)
        return 0

    lax.fori_loop(0, GROUPS // 2, body, 0)
    pltpu.async_copy(out_v, out_t_hbm.at[:, pl.ds(base, B_PER_W)], sem_a).wait()


def kernel(inputs, table):
    idx = inputs.astype(jnp.int32)
    out_t = _embed_gather_t(idx, table.T)
    return out_t.T
